# baseline (speedup 1.0000x reference)
import jax
import jax.numpy as jnp
from jax import lax
from jax.experimental import pallas as pl
from jax.experimental.pallas import tpu as pltpu
from jax.experimental.pallas import tpu_sc as plsc

N = 100000
E = 3200000
L = 16
NC, NS = 2, 16
NW = NC * NS
NP = 100352
ROWS = E // 128
G = 8
NCH = ROWS // G
D = 4

C1 = 58
C0 = (NCH - 16 * C1) // 16
C0_EXTRA = NCH - 16 * C1 - 16 * C0
C1_BASE = 16 * C0 + C0_EXTRA

PNT = NP // NW
XCH = 6 * PNT

_mesh = plsc.VectorSubcoreMesh(
    core_axis_name="c", subcore_axis_name="s", num_cores=NC, num_subcores=NS
)


def _make_edge_pass(gather: bool):
    scratch = [
        pltpu.VMEM_SHARED((NP,), jnp.float32),
        pltpu.VMEM((D, G, 128), jnp.int32),
        pltpu.VMEM((D, G, 128), jnp.float32),
        [pltpu.SemaphoreType.DMA] * D,
        [pltpu.SemaphoreType.DMA] * D,
    ]
    if gather:
        scratch.append(pltpu.VMEM((NP,), jnp.float32))
        scratch.append(pltpu.VMEM((D, G, 128), jnp.int32))
    else:
        scratch.append(pltpu.VMEM((XCH,), jnp.float32))
        scratch.append(pltpu.VMEM((PNT,), jnp.float32))
        scratch.append(pltpu.VMEM((6, 16), jnp.float32))
        scratch.append(pltpu.VMEM((16,), jnp.float32))

    def body(*refs):
        if gather:
            (q_hbm, src_hbm, dst_hbm, zero_hbm, out_hbm,
             acc_sh, dst_v, vals_v, lsem, ssem, q_v, src_v) = refs
        else:
            (dst_hbm, xf_hbm, w1_hbm, w2_hbm, zero_hbm, out_hbm, p_hbm,
             acc_sh, dst_v, vals_v, lsem, ssem, x_v, p_v, w1_v, w2_v) = refs

        cid = lax.axis_index("c")
        sid = lax.axis_index("s")
        wid = sid * NC + cid
        n_chunk = jnp.where(cid == 0,
                            C0 + jnp.where(sid < C0_EXTRA, 1, 0), C1)
        bc = jnp.where(cid == 0,
                       sid * C0 + jnp.minimum(sid, C0_EXTRA),
                       C1_BASE + sid * C1)

        @pl.when(sid == 0)
        def _zero():
            pltpu.sync_copy(zero_hbm, acc_sh)

        def fire_load(k, b):
            r0 = jnp.minimum((bc + k) * G, ROWS - G)
            pltpu.async_copy(dst_hbm.at[pl.ds(r0, G)], dst_v.at[b], lsem[b])
            if gather:
                pltpu.async_copy(src_hbm.at[pl.ds(r0, G)], src_v.at[b], lsem[b])

        def wait_load(b):
            pltpu.make_async_copy(dst_hbm.at[pl.ds(0, G)], dst_v.at[b],
                                  lsem[b]).wait()
            if gather:
                pltpu.make_async_copy(src_hbm.at[pl.ds(0, G)], src_v.at[b],
                                      lsem[b]).wait()

        def do_gather(b):
            if gather:
                for j in range(G):
                    for c in range(128 // L):
                        idx = src_v[b, j, pl.ds(c * L, L)]
                        vals_v[b, j, pl.ds(c * L, L)] = plsc.load_gather(
                            q_v, [idx])

        def fire_scatter(b):
            for j in range(G):
                pltpu.async_copy(vals_v.at[b].at[j],
                                 acc_sh.at[dst_v.at[b].at[j]],
                                 ssem[b], add=True)

        def drain_scatter(b):
            for j in range(G):
                pltpu.make_async_copy(vals_v.at[b].at[j],
                                      acc_sh.at[dst_v.at[b].at[j]],
                                      ssem[b]).wait()

        if gather:
            pltpu.sync_copy(q_hbm, q_v)
        else:
            ones = jnp.full((L,), 1.0, dtype=jnp.float32)
            for b in range(D):
                for j in range(G):
                    for c in range(128 // L):
                        vals_v[b, j, pl.ds(c * L, L)] = ones

        plsc.subcore_barrier()

        fire_load(0, 0)
        fire_load(1, 1)

        if not gather:
            n0 = wid * PNT
            s0 = jnp.minimum(6 * n0, 6 * N - XCH)
            delta = 6 * n0 - s0
            pltpu.sync_copy(xf_hbm.at[pl.ds(s0, XCH)], x_v)
            pltpu.sync_copy(w1_hbm, w1_v)
            pltpu.sync_copy(w2_hbm, w2_v)
            w2c = w2_v[...]
            wk = [jnp.sum(w1_v[k, :] * w2c) for k in range(6)]

            def p_body(i, carry):
                li6 = (i * 96 + delta) + lax.iota(jnp.int32, 16) * 6
                acc = jnp.zeros((L,), jnp.float32)
                for k in range(6):
                    idx = jnp.minimum(li6 + k, XCH - 1)
                    acc = acc + plsc.load_gather(x_v, [idx]) * wk[k]
                p_v[pl.ds(i * L, L)] = acc
                return carry

            lax.fori_loop(0, PNT // L, p_body, 0)
            pltpu.sync_copy(p_v, p_hbm.at[pl.ds(n0, PNT)])

        def chunk_body(m, carry):
            for b in range(D):
                k = m * D + b
                wait_load(b)
                do_gather(b)
                fire_scatter(b)
                b2 = (b + 2) % D
                @pl.when(k >= 2)
                def _():
                    drain_scatter(b2)
                fire_load(k + 2, b2)
            return carry

        lax.fori_loop(0, n_chunk // D, chunk_body, 0)

        drain_scatter(2)
        drain_scatter(3)
        wait_load(0)
        wait_load(1)

        def tail_body(i, carry):
            k = (n_chunk // D) * D + i
            fire_load(k, 0)
            wait_load(0)
            do_gather(0)
            fire_scatter(0)
            drain_scatter(0)
            return carry

        lax.fori_loop(0, n_chunk % D, tail_body, 0)

        plsc.subcore_barrier()

        @pl.when(sid == 0)
        def _writeout():
            pltpu.sync_copy(acc_sh, out_hbm.at[cid])

    if gather:
        out_type = jax.ShapeDtypeStruct((NC, NP), jnp.float32)
    else:
        out_type = (jax.ShapeDtypeStruct((NC, NP), jnp.float32),
                    jax.ShapeDtypeStruct((NP,), jnp.float32))
    return pl.kernel(
        body,
        out_type=out_type,
        mesh=_mesh,
        scratch_types=scratch,
        compiler_params=pltpu.CompilerParams(needs_layout_passes=False),
    )


_deg_pass = _make_edge_pass(gather=False)
_msg_pass = _make_edge_pass(gather=True)


def _stage1_body(degp_ref, p_ref, dinv_ref, q1_ref):
    deg = degp_ref[0] + degp_ref[1] + 1.0
    dinv = lax.rsqrt(deg)
    dinv_ref[...] = dinv
    q1_ref[...] = dinv * p_ref[...]


def _stage2_body(s1p_ref, dinv_ref, p_ref, b1_ref, w2_ref, h2_ref, q2_ref):
    dinv = dinv_ref[...]
    c1 = jnp.sum(b1_ref[...] * w2_ref[...])
    h2 = dinv * (s1p_ref[0] + s1p_ref[1]) + dinv * dinv * p_ref[...] + c1
    h2_ref[...] = h2
    q2_ref[...] = dinv * h2


def _stage3_body(s2p_ref, dinv_ref, h2_ref, b2_ref, out_ref):
    dinv = dinv_ref[...]
    o = (dinv * (s2p_ref[0] + s2p_ref[1])
         + dinv * dinv * h2_ref[...] + b2_ref[0])
    out_ref[...] = jnp.maximum(o, 0.01 * o)


_stage1 = pl.pallas_call(
    _stage1_body,
    out_shape=[jax.ShapeDtypeStruct((NP,), jnp.float32)] * 2,
)
_stage2 = pl.pallas_call(
    _stage2_body,
    out_shape=[jax.ShapeDtypeStruct((NP,), jnp.float32)] * 2,
)
_stage3 = pl.pallas_call(
    _stage3_body,
    out_shape=jax.ShapeDtypeStruct((NP,), jnp.float32),
)


@jax.jit
def kernel(x, edge_index, W1, b1, W2, b2):
    e3d = edge_index.reshape(2, ROWS, 128)
    src = e3d[0]
    dst = e3d[1]
    x_flat = x.reshape(-1)
    w2c = W2.reshape(-1)
    zero = jnp.zeros((NP,), jnp.float32)

    degp, p = _deg_pass(dst, x_flat, W1, w2c, zero)
    dinv, q1 = _stage1(degp, p)
    s1p = _msg_pass(q1, src, dst, zero)
    h2, q2 = _stage2(s1p, dinv, p, b1, w2c)
    s2p = _msg_pass(q2, src, dst, zero)
    out = _stage3(s2p, dinv, h2, b2)
    return out[:N]


# speedup vs baseline: 1.0513x; 1.0513x over previous
import jax
import jax.numpy as jnp
from jax import lax
from jax.experimental import pallas as pl
from jax.experimental.pallas import tpu as pltpu
from jax.experimental.pallas import tpu_sc as plsc

N = 100000
E = 3200000
L = 16
NC, NS = 2, 16
NW = NC * NS
NP = 100352
ROWS = E // 128
G = 8
NCH = ROWS // G
D = 4

C1 = 58
C0 = (NCH - 16 * C1) // 16
C0_EXTRA = NCH - 16 * C1 - 16 * C0
C1_BASE = 16 * C0 + C0_EXTRA

PNT = NP // NW
XCH = 6 * PNT

_mesh = plsc.VectorSubcoreMesh(
    core_axis_name="c", subcore_axis_name="s", num_cores=NC, num_subcores=NS
)


def _make_edge_pass(gather: bool):
    scratch = [
        pltpu.VMEM_SHARED((NP,), jnp.float32),
        pltpu.VMEM((D, G, 128), jnp.int32),
        pltpu.VMEM((D, G, 128), jnp.float32),
        [pltpu.SemaphoreType.DMA] * D,
        [pltpu.SemaphoreType.DMA] * D,
    ]
    if gather:
        scratch.append(pltpu.VMEM((NP,), jnp.float32))
        scratch.append(pltpu.VMEM((D, G, 128), jnp.int32))
    else:
        scratch.append(pltpu.VMEM((XCH,), jnp.float32))
        scratch.append(pltpu.VMEM((PNT,), jnp.float32))
        scratch.append(pltpu.VMEM((6, 16), jnp.float32))
        scratch.append(pltpu.VMEM((16,), jnp.float32))

    def body(*refs):
        if gather:
            (q_hbm, e_hbm, zero_hbm, out_hbm,
             acc_sh, dst_v, vals_v, lsem, ssem, q_v, src_v) = refs
        else:
            (e_hbm, xf_hbm, w1_hbm, w2_hbm, zero_hbm, out_hbm, p_hbm,
             acc_sh, dst_v, vals_v, lsem, ssem, x_v, p_v, w1_v, w2_v) = refs
        src_hbm = e_hbm.at[0]
        dst_hbm = e_hbm.at[1]

        cid = lax.axis_index("c")
        sid = lax.axis_index("s")
        wid = sid * NC + cid
        n_chunk = jnp.where(cid == 0,
                            C0 + jnp.where(sid < C0_EXTRA, 1, 0), C1)
        bc = jnp.where(cid == 0,
                       sid * C0 + jnp.minimum(sid, C0_EXTRA),
                       C1_BASE + sid * C1)

        @pl.when(sid == 0)
        def _zero():
            pltpu.sync_copy(zero_hbm, acc_sh)

        def fire_load(k, b):
            r0 = jnp.minimum((bc + k) * G, ROWS - G)
            pltpu.async_copy(dst_hbm.at[pl.ds(r0, G)], dst_v.at[b], lsem[b])
            if gather:
                pltpu.async_copy(src_hbm.at[pl.ds(r0, G)], src_v.at[b], lsem[b])

        def wait_load(b):
            pltpu.make_async_copy(dst_hbm.at[pl.ds(0, G)], dst_v.at[b],
                                  lsem[b]).wait()
            if gather:
                pltpu.make_async_copy(src_hbm.at[pl.ds(0, G)], src_v.at[b],
                                      lsem[b]).wait()

        def do_gather(b):
            if gather:
                for j in range(G):
                    for c in range(128 // L):
                        idx = src_v[b, j, pl.ds(c * L, L)]
                        vals_v[b, j, pl.ds(c * L, L)] = plsc.load_gather(
                            q_v, [idx])

        def fire_scatter(b):
            for j in range(G):
                pltpu.async_copy(vals_v.at[b].at[j],
                                 acc_sh.at[dst_v.at[b].at[j]],
                                 ssem[b], add=True)

        def drain_scatter(b):
            for j in range(G):
                pltpu.make_async_copy(vals_v.at[b].at[j],
                                      acc_sh.at[dst_v.at[b].at[j]],
                                      ssem[b]).wait()

        if gather:
            pltpu.sync_copy(q_hbm, q_v)
        else:
            ones = jnp.full((L,), 1.0, dtype=jnp.float32)
            for b in range(D):
                for j in range(G):
                    for c in range(128 // L):
                        vals_v[b, j, pl.ds(c * L, L)] = ones

        plsc.subcore_barrier()

        fire_load(0, 0)
        fire_load(1, 1)

        if not gather:
            n0 = wid * PNT
            s0 = jnp.minimum(6 * n0, 6 * N - XCH)
            delta = 6 * n0 - s0
            pltpu.sync_copy(xf_hbm.at[pl.ds(s0, XCH)], x_v)
            pltpu.sync_copy(w1_hbm, w1_v)
            pltpu.sync_copy(w2_hbm, w2_v)
            w2c = w2_v[...]
            wk = [jnp.sum(w1_v[k, :] * w2c) for k in range(6)]

            def p_body(i, carry):
                li6 = (i * 96 + delta) + lax.iota(jnp.int32, 16) * 6
                acc = jnp.zeros((L,), jnp.float32)
                for k in range(6):
                    idx = jnp.minimum(li6 + k, XCH - 1)
                    acc = acc + plsc.load_gather(x_v, [idx]) * wk[k]
                p_v[pl.ds(i * L, L)] = acc
                return carry

            lax.fori_loop(0, PNT // L, p_body, 0)
            pltpu.sync_copy(p_v, p_hbm.at[pl.ds(n0, PNT)])

        def chunk_body(m, carry):
            for b in range(D):
                k = m * D + b
                wait_load(b)
                do_gather(b)
                fire_scatter(b)
                b2 = (b + 2) % D
                @pl.when(k >= 2)
                def _():
                    drain_scatter(b2)
                fire_load(k + 2, b2)
            return carry

        lax.fori_loop(0, n_chunk // D, chunk_body, 0)

        drain_scatter(2)
        drain_scatter(3)
        wait_load(0)
        wait_load(1)

        def tail_body(i, carry):
            k = (n_chunk // D) * D + i
            fire_load(k, 0)
            wait_load(0)
            do_gather(0)
            fire_scatter(0)
            drain_scatter(0)
            return carry

        lax.fori_loop(0, n_chunk % D, tail_body, 0)

        plsc.subcore_barrier()

        @pl.when(sid == 0)
        def _writeout():
            pltpu.sync_copy(acc_sh, out_hbm.at[cid])

    if gather:
        out_type = jax.ShapeDtypeStruct((NC, NP), jnp.float32)
    else:
        out_type = (jax.ShapeDtypeStruct((NC, NP), jnp.float32),
                    jax.ShapeDtypeStruct((NP,), jnp.float32))
    return pl.kernel(
        body,
        out_type=out_type,
        mesh=_mesh,
        scratch_types=scratch,
        compiler_params=pltpu.CompilerParams(needs_layout_passes=False),
    )


_deg_pass = _make_edge_pass(gather=False)
_msg_pass = _make_edge_pass(gather=True)


def _stage1_body(degp_ref, p_ref, dinv_ref, q1_ref):
    deg = degp_ref[0] + degp_ref[1] + 1.0
    dinv = lax.rsqrt(deg)
    dinv_ref[...] = dinv
    q1_ref[...] = dinv * p_ref[...]


def _stage2_body(s1p_ref, dinv_ref, p_ref, b1_ref, w2_ref, h2_ref, q2_ref):
    dinv = dinv_ref[...]
    c1 = jnp.sum(b1_ref[...] * w2_ref[...])
    h2 = dinv * (s1p_ref[0] + s1p_ref[1]) + dinv * dinv * p_ref[...] + c1
    h2_ref[...] = h2
    q2_ref[...] = dinv * h2


def _stage3_body(s2p_ref, dinv_ref, h2_ref, b2_ref, out_ref):
    dinv = dinv_ref[...]
    o = (dinv * (s2p_ref[0] + s2p_ref[1])
         + dinv * dinv * h2_ref[...] + b2_ref[0])
    out_ref[...] = jnp.maximum(o, 0.01 * o)


_stage1 = pl.pallas_call(
    _stage1_body,
    out_shape=[jax.ShapeDtypeStruct((NP,), jnp.float32)] * 2,
)
_stage2 = pl.pallas_call(
    _stage2_body,
    out_shape=[jax.ShapeDtypeStruct((NP,), jnp.float32)] * 2,
)
_stage3 = pl.pallas_call(
    _stage3_body,
    out_shape=jax.ShapeDtypeStruct((NP,), jnp.float32),
)


@jax.jit
def kernel(x, edge_index, W1, b1, W2, b2):
    e3d = edge_index.reshape(2, ROWS, 128)
    x_flat = x.reshape(-1)
    w2c = W2.reshape(-1)
    zero = jnp.zeros((NP,), jnp.float32)

    degp, p = _deg_pass(e3d, x_flat, W1, w2c, zero)
    dinv, q1 = _stage1(degp, p)
    s1p = _msg_pass(q1, e3d, zero)
    h2, q2 = _stage2(s1p, dinv, p, b1, w2c)
    s2p = _msg_pass(q2, e3d, zero)
    out = _stage3(s2p, dinv, h2, b2)
    return out[:N]
